# baseline (speedup 1.0000x reference)
import jax
import jax.numpy as jnp
from jax import lax
from jax.experimental import pallas as pl
from jax.experimental.pallas import tpu as pltpu
from jax.experimental.pallas import tpu_sc as plsc

VOCAB = 1000000
MAXLEN = 200
EMBED = 32
BATCH = 4096

NC, NS, L = 2, 16, 16
NW = NC * NS
BT = BATCH // NW
S_PER = 4
NGRP = MAXLEN // S_PER
ET = EMBED // 8


def _body(x_hbm, tab_hbm, posx_hbm, out_hbm,
          idx0, idx1, stag0, stag1, obuf0, obuf1, posb0, posb1,
          si0, si1, sg0, sg1, ss0, ss1, sp0, sp1):
    idxs = (idx0, idx1)
    stags = (stag0, stag1)
    obufs = (obuf0, obuf1)
    posbs = (posb0, posb1)
    sem_i = (si0, si1)
    sem_g = (sg0, sg1)
    sem_s = (ss0, ss1)
    sem_p = (sp0, sp1)

    w = lax.axis_index("s") * NC + lax.axis_index("c")
    bcol = w * BT
    iota = lax.iota(jnp.int32, L)

    for b in range(2):
        pltpu.async_copy(
            x_hbm.at[pl.ds(b * S_PER, S_PER), pl.ds(bcol, BT)],
            idxs[b], sem_i[b])
        pltpu.async_copy(
            posx_hbm.at[pl.ds(b * S_PER, S_PER)], posbs[b], sem_p[b])

    def pair_body(it, carry):
        for b in range(2):
            g = it * 2 + b
            s0 = g * S_PER

            @pl.when(it > 0)
            def _():
                pltpu.make_async_copy(
                    obufs[b],
                    out_hbm.at[pl.ds(s0 - 2 * S_PER, S_PER), pl.ds(0, ET), w],
                    sem_s[b]).wait()

            pltpu.make_async_copy(
                x_hbm.at[pl.ds(s0, S_PER), pl.ds(bcol, BT)],
                idxs[b], sem_i[b]).wait()
            pltpu.make_async_copy(
                posx_hbm.at[pl.ds(s0, S_PER)], posbs[b], sem_p[b]).wait()

            descs = []
            for j in range(S_PER):
                descs.append(pltpu.async_copy(
                    tab_hbm.at[idxs[b].at[j]],
                    stags[b].at[pl.ds(j * BT, BT)],
                    sem_g[b]))
            for d in descs:
                d.wait()

            def trans_body(ss, carry2):
                rbase = ss * BT
                for e in range(EMBED):
                    pv = posbs[b][ss, pl.ds(e * L, L)]
                    colidx = jnp.full((L,), e, jnp.int32)
                    for v in range(BT // L):
                        rowidx = iota + (rbase + v * L)
                        val = plsc.load_gather(stags[b], [rowidx, colidx])
                        obufs[b][ss, e // 8, e % 8, pl.ds(v * L, L)] = val + pv
                return carry2

            lax.fori_loop(0, S_PER, trans_body, 0)

            pltpu.async_copy(
                obufs[b],
                out_hbm.at[pl.ds(s0, S_PER), pl.ds(0, ET), w],
                sem_s[b])

            @pl.when(g + 2 < NGRP)
            def _():
                pltpu.async_copy(
                    x_hbm.at[pl.ds(s0 + 2 * S_PER, S_PER), pl.ds(bcol, BT)],
                    idxs[b], sem_i[b])
                pltpu.async_copy(
                    posx_hbm.at[pl.ds(s0 + 2 * S_PER, S_PER)],
                    posbs[b], sem_p[b])
        return carry

    lax.fori_loop(0, NGRP // 2, pair_body, 0)

    for b in range(2):
        s0 = (NGRP - 2 + b) * S_PER
        pltpu.make_async_copy(
            obufs[b],
            out_hbm.at[pl.ds(s0, S_PER), pl.ds(0, ET), w],
            sem_s[b]).wait()


def kernel(x, token_table, pos_table):
    xT = x.astype(jnp.int32).T
    posx = jnp.repeat(pos_table, L, axis=1)
    mesh = plsc.VectorSubcoreMesh(core_axis_name="c", subcore_axis_name="s",
                                  num_cores=NC, num_subcores=NS)
    k = pl.kernel(
        _body,
        out_type=jax.ShapeDtypeStruct((MAXLEN, ET, NW, 8, BT), jnp.float32),
        mesh=mesh,
        scratch_types=[
            pltpu.VMEM((S_PER, BT), jnp.int32),
            pltpu.VMEM((S_PER, BT), jnp.int32),
            pltpu.VMEM((S_PER * BT, EMBED), jnp.float32),
            pltpu.VMEM((S_PER * BT, EMBED), jnp.float32),
            pltpu.VMEM((S_PER, ET, 8, BT), jnp.float32),
            pltpu.VMEM((S_PER, ET, 8, BT), jnp.float32),
            pltpu.VMEM((S_PER, EMBED * L), jnp.float32),
            pltpu.VMEM((S_PER, EMBED * L), jnp.float32),
            pltpu.SemaphoreType.DMA,
            pltpu.SemaphoreType.DMA,
            pltpu.SemaphoreType.DMA,
            pltpu.SemaphoreType.DMA,
            pltpu.SemaphoreType.DMA,
            pltpu.SemaphoreType.DMA,
            pltpu.SemaphoreType.DMA,
            pltpu.SemaphoreType.DMA,
        ],
        compiler_params=pltpu.CompilerParams(use_tc_tiling_on_sc=False,
                                             needs_layout_passes=False),
    )
    out5 = k(xT, token_table, posx)
    return out5.transpose(2, 4, 0, 1, 3).reshape(BATCH, MAXLEN, EMBED)


# speedup vs baseline: 1.0301x; 1.0301x over previous
import jax
import jax.numpy as jnp
from jax import lax
from jax.experimental import pallas as pl
from jax.experimental.pallas import tpu as pltpu
from jax.experimental.pallas import tpu_sc as plsc

VOCAB = 1000000
MAXLEN = 200
EMBED = 32
BATCH = 4096

NC, NS, L = 2, 16, 16
NW = NC * NS
BT = BATCH // NW
S_PER = 4
NGRP = MAXLEN // S_PER
ET = EMBED // 8


def _body(x_hbm, tab_hbm, posx_hbm, out_hbm,
          idx0, idx1, stag0, stag1, obuf0, obuf1, posb0, posb1,
          si0, si1, sg0, sg1, ss0, ss1, sp0, sp1):
    idxs = (idx0, idx1)
    stags = (stag0, stag1)
    obufs = (obuf0, obuf1)
    posbs = (posb0, posb1)
    sem_i = (si0, si1)
    sem_g = (sg0, sg1)
    sem_s = (ss0, ss1)
    sem_p = (sp0, sp1)

    w = lax.axis_index("s") * NC + lax.axis_index("c")
    bcol = w * BT
    iota = lax.iota(jnp.int32, L)

    def fire_idx(b, g):
        pltpu.async_copy(
            x_hbm.at[pl.ds(g * S_PER, S_PER), pl.ds(bcol, BT)],
            idxs[b], sem_i[b])
        pltpu.async_copy(
            posx_hbm.at[pl.ds(g * S_PER, S_PER)], posbs[b], sem_p[b])

    def wait_idx(b, g):
        pltpu.make_async_copy(
            x_hbm.at[pl.ds(g * S_PER, S_PER), pl.ds(bcol, BT)],
            idxs[b], sem_i[b]).wait()
        pltpu.make_async_copy(
            posx_hbm.at[pl.ds(g * S_PER, S_PER)], posbs[b], sem_p[b]).wait()

    def fire_gathers(b):
        for j in range(S_PER):
            pltpu.async_copy(
                tab_hbm.at[idxs[b].at[j]],
                stags[b].at[pl.ds(j * BT, BT)],
                sem_g[b])

    def wait_gathers(b):
        for j in range(S_PER):
            pltpu.make_async_copy(
                tab_hbm.at[idxs[b].at[j]],
                stags[b].at[pl.ds(j * BT, BT)],
                sem_g[b]).wait()

    def out_slice(g):
        return out_hbm.at[pl.ds(g * S_PER, S_PER), pl.ds(0, ET), w]

    def transpose_group(b):
        def trans_body(ss, carry2):
            rbase = ss * BT
            ridx = [iota + (rbase + v * L) for v in range(BT // L)]
            for e in range(EMBED):
                pv = posbs[b][ss, pl.ds(e * L, L)]
                colidx = jnp.full((L,), e, jnp.int32)
                for v in range(BT // L):
                    val = plsc.load_gather(stags[b], [ridx[v], colidx])
                    obufs[b][ss, e // 8, e % 8, pl.ds(v * L, L)] = val + pv
            return carry2

        lax.fori_loop(0, S_PER, trans_body, 0)

    fire_idx(0, 0)
    fire_idx(1, 1)
    wait_idx(0, 0)
    fire_gathers(0)

    def pair_body(it, carry):
        for b in range(2):
            g = it * 2 + b
            o = b ^ 1

            @pl.when(g + 1 < NGRP)
            def _():
                wait_idx(o, g + 1)
                fire_gathers(o)

            wait_gathers(b)

            @pl.when(it > 0)
            def _():
                pltpu.make_async_copy(obufs[b], out_slice(g - 2), sem_s[b]).wait()

            transpose_group(b)
            pltpu.async_copy(obufs[b], out_slice(g), sem_s[b])

            @pl.when(g + 2 < NGRP)
            def _():
                fire_idx(b, g + 2)
        return carry

    lax.fori_loop(0, NGRP // 2, pair_body, 0)

    for b in range(2):
        pltpu.make_async_copy(
            obufs[b], out_slice(NGRP - 2 + b), sem_s[b]).wait()


def kernel(x, token_table, pos_table):
    xT = x.astype(jnp.int32).T
    posx = jnp.repeat(pos_table, L, axis=1)
    mesh = plsc.VectorSubcoreMesh(core_axis_name="c", subcore_axis_name="s",
                                  num_cores=NC, num_subcores=NS)
    k = pl.kernel(
        _body,
        out_type=jax.ShapeDtypeStruct((MAXLEN, ET, NW, 8, BT), jnp.float32),
        mesh=mesh,
        scratch_types=[
            pltpu.VMEM((S_PER, BT), jnp.int32),
            pltpu.VMEM((S_PER, BT), jnp.int32),
            pltpu.VMEM((S_PER * BT, EMBED), jnp.float32),
            pltpu.VMEM((S_PER * BT, EMBED), jnp.float32),
            pltpu.VMEM((S_PER, ET, 8, BT), jnp.float32),
            pltpu.VMEM((S_PER, ET, 8, BT), jnp.float32),
            pltpu.VMEM((S_PER, EMBED * L), jnp.float32),
            pltpu.VMEM((S_PER, EMBED * L), jnp.float32),
            pltpu.SemaphoreType.DMA,
            pltpu.SemaphoreType.DMA,
            pltpu.SemaphoreType.DMA,
            pltpu.SemaphoreType.DMA,
            pltpu.SemaphoreType.DMA,
            pltpu.SemaphoreType.DMA,
            pltpu.SemaphoreType.DMA,
            pltpu.SemaphoreType.DMA,
        ],
        compiler_params=pltpu.CompilerParams(use_tc_tiling_on_sc=False,
                                             needs_layout_passes=False),
    )
    out5 = k(xT, token_table, posx)
    return out5.transpose(2, 4, 0, 1, 3).reshape(BATCH, MAXLEN, EMBED)


# speedup vs baseline: 1.4536x; 1.4110x over previous
import jax
import jax.numpy as jnp
from jax import lax
from jax.experimental import pallas as pl
from jax.experimental.pallas import tpu as pltpu
from jax.experimental.pallas import tpu_sc as plsc

VOCAB = 1000000
MAXLEN = 200
EMBED = 32
BATCH = 4096

NC, NS, L = 2, 16, 16
NW = NC * NS
BT = BATCH // NW
S_PER = 4
NGRP = MAXLEN // S_PER
ET = EMBED // 8


def _body(x_hbm, tab_hbm, posx_hbm, out_hbm,
          idx0, idx1, stag0, stag1, obuf0, obuf1, posb0, posb1,
          si0, si1, sg0, sg1, ss0, ss1, sp0, sp1):
    idxs = (idx0, idx1)
    stags = (stag0, stag1)
    obufs = (obuf0, obuf1)
    posbs = (posb0, posb1)
    sem_i = (si0, si1)
    sem_g = (sg0, sg1)
    sem_s = (ss0, ss1)
    sem_p = (sp0, sp1)

    w = lax.axis_index("s") * NC + lax.axis_index("c")
    bcol = w * BT
    iota = lax.iota(jnp.int32, L)

    def fire_idx(b, g):
        pltpu.async_copy(
            x_hbm.at[pl.ds(g * S_PER, S_PER), pl.ds(bcol, BT)],
            idxs[b], sem_i[b])
        pltpu.async_copy(
            posx_hbm.at[pl.ds(g * S_PER, S_PER)], posbs[b], sem_p[b])

    def wait_idx(b, g):
        pltpu.make_async_copy(
            x_hbm.at[pl.ds(g * S_PER, S_PER), pl.ds(bcol, BT)],
            idxs[b], sem_i[b]).wait()
        pltpu.make_async_copy(
            posx_hbm.at[pl.ds(g * S_PER, S_PER)], posbs[b], sem_p[b]).wait()

    def fire_gathers(b):
        for j in range(S_PER):
            pltpu.async_copy(
                tab_hbm.at[idxs[b].at[j]],
                stags[b].at[pl.ds(j * BT, BT)],
                sem_g[b])

    def wait_gathers(b):
        for j in range(S_PER):
            pltpu.make_async_copy(
                tab_hbm.at[idxs[b].at[j]],
                stags[b].at[pl.ds(j * BT, BT)],
                sem_g[b]).wait()

    def out_slice(g):
        return out_hbm.at[pl.ds(g * S_PER, S_PER), pl.ds(0, ET), w]

    def transpose_group(b):
        def trans_body(ss, carry2):
            rbase = ss * BT
            ss_v = jnp.full((L,), 0, jnp.int32) + ss
            for h in range(EMBED // L):
                e_vec = iota + h * L
                te_c = lax.shift_right_logical(e_vec, 3)
                e8_c = lax.bitwise_and(e_vec, 7)
                pv = posbs[b][ss, pl.ds(h * L, L)]
                for kk in range(BT):
                    kk_v = jnp.full((L,), kk, jnp.int32)
                    val = stags[b][rbase + kk, pl.ds(h * L, L)] + pv
                    plsc.store_scatter(obufs[b], [ss_v, te_c, e8_c, kk_v], val)
            return carry2

        lax.fori_loop(0, S_PER, trans_body, 0)

    fire_idx(0, 0)
    fire_idx(1, 1)
    wait_idx(0, 0)
    fire_gathers(0)

    def pair_body(it, carry):
        for b in range(2):
            g = it * 2 + b
            o = b ^ 1

            @pl.when(g + 1 < NGRP)
            def _():
                wait_idx(o, g + 1)
                fire_gathers(o)

            wait_gathers(b)

            @pl.when(it > 0)
            def _():
                pltpu.make_async_copy(
                    obufs[b].at[:, :, :, pl.ds(0, BT)], out_slice(g - 2),
                    sem_s[b]).wait()

            transpose_group(b)
            pltpu.async_copy(
                obufs[b].at[:, :, :, pl.ds(0, BT)], out_slice(g), sem_s[b])

            @pl.when(g + 2 < NGRP)
            def _():
                fire_idx(b, g + 2)
        return carry

    lax.fori_loop(0, NGRP // 2, pair_body, 0)

    for b in range(2):
        pltpu.make_async_copy(
            obufs[b].at[:, :, :, pl.ds(0, BT)], out_slice(NGRP - 2 + b),
            sem_s[b]).wait()


def kernel(x, token_table, pos_table):
    xT = x.astype(jnp.int32).T
    mesh = plsc.VectorSubcoreMesh(core_axis_name="c", subcore_axis_name="s",
                                  num_cores=NC, num_subcores=NS)
    k = pl.kernel(
        _body,
        out_type=jax.ShapeDtypeStruct((MAXLEN, ET, NW, 8, BT), jnp.float32),
        mesh=mesh,
        scratch_types=[
            pltpu.VMEM((S_PER, BT), jnp.int32),
            pltpu.VMEM((S_PER, BT), jnp.int32),
            pltpu.VMEM((S_PER * BT, EMBED), jnp.float32),
            pltpu.VMEM((S_PER * BT, EMBED), jnp.float32),
            pltpu.VMEM((S_PER, ET, 8, BT + 1), jnp.float32),
            pltpu.VMEM((S_PER, ET, 8, BT + 1), jnp.float32),
            pltpu.VMEM((S_PER, EMBED), jnp.float32),
            pltpu.VMEM((S_PER, EMBED), jnp.float32),
            pltpu.SemaphoreType.DMA,
            pltpu.SemaphoreType.DMA,
            pltpu.SemaphoreType.DMA,
            pltpu.SemaphoreType.DMA,
            pltpu.SemaphoreType.DMA,
            pltpu.SemaphoreType.DMA,
            pltpu.SemaphoreType.DMA,
            pltpu.SemaphoreType.DMA,
        ],
        compiler_params=pltpu.CompilerParams(use_tc_tiling_on_sc=False,
                                             needs_layout_passes=False),
    )
    out5 = k(xT, token_table, pos_table)
    return out5.transpose(2, 4, 0, 1, 3).reshape(BATCH, MAXLEN, EMBED)
